# baseline (speedup 1.0000x reference)
import functools

import jax
import jax.numpy as jnp
from jax import lax
from jax.experimental import pallas as pl
from jax.experimental.pallas import tpu as pltpu
from jax.experimental.pallas import tpu_sc as plsc

_NUM_REGIONS = 6
_EMB_DIM = 1024
_BATCH = 4096
_ROWS = _BATCH * _NUM_REGIONS
_NC = 2
_NS = 16
_NW = _NC * _NS
_ROWS_PER_W = _ROWS // _NW
_COPIES = 16
_STAGE_ROWS = _COPIES * _NUM_REGIONS
_SCATTERS = _ROWS_PER_W // _STAGE_ROWS


def _sc_body(table_hbm, idx_hbm, out_hbm, idx_v, stage_v, sem):
    wid = lax.axis_index("c") * _NS + lax.axis_index("s")
    base = wid * _ROWS_PER_W
    pltpu.sync_copy(idx_hbm, idx_v)
    pltpu.async_copy(table_hbm.at[idx_v], stage_v, sem).wait()
    handles = [
        pltpu.async_copy(
            stage_v,
            out_hbm.at[pl.ds(base + k * _STAGE_ROWS, _STAGE_ROWS)],
            sem,
        )
        for k in range(_SCATTERS)
    ]
    for h in handles:
        h.wait()


def kernel(batch_size, table, region_ids):
    delta = jnp.asarray(batch_size - _BATCH, jnp.float32)
    table_pa = table.astype(jnp.float32) + delta
    idx = jnp.tile(region_ids.astype(jnp.int32), _COPIES)

    mesh = plsc.VectorSubcoreMesh(core_axis_name="c", subcore_axis_name="s")
    run = functools.partial(
        pl.kernel,
        mesh=mesh,
        out_type=jax.ShapeDtypeStruct((_ROWS, _EMB_DIM), jnp.float32),
        scratch_types=[
            pltpu.VMEM((_STAGE_ROWS,), jnp.int32),
            pltpu.VMEM((_STAGE_ROWS, _EMB_DIM), jnp.float32),
            pltpu.SemaphoreType.DMA,
        ],
    )(_sc_body)
    out = run(table_pa, idx)
    return out.reshape(_BATCH, _NUM_REGIONS, _EMB_DIM)


# speedup vs baseline: 1.6057x; 1.6057x over previous
import functools

import jax
import jax.numpy as jnp
from jax import lax
from jax.experimental import pallas as pl
from jax.experimental.pallas import tpu as pltpu
from jax.experimental.pallas import tpu_sc as plsc

_NUM_REGIONS = 6
_EMB_DIM = 1024
_BATCH = 4096
_NC = 2
_NS = 16
_NW = _NC * _NS
_B_PER_W = _BATCH // _NW
_INFLIGHT = 16
_IDX_PAD = 16


def _sc_body(table_hbm, idx_hbm, out_hbm, idx_v, tokens_v, sem):
    wid = lax.axis_index("c") * _NS + lax.axis_index("s")
    base = wid * _B_PER_W
    pltpu.sync_copy(idx_hbm, idx_v)
    pltpu.async_copy(table_hbm.at[idx_v], tokens_v, sem).wait()
    src = tokens_v.at[pl.ds(0, _NUM_REGIONS)]

    def step(i, carry):
        b0 = base + i * _INFLIGHT
        handles = [
            pltpu.async_copy(src, out_hbm.at[b0 + k], sem)
            for k in range(_INFLIGHT)
        ]
        for h in handles:
            h.wait()
        return carry

    lax.fori_loop(0, _B_PER_W // _INFLIGHT, step, 0)


def kernel(batch_size, table, region_ids):
    delta = jnp.asarray(batch_size - _BATCH, jnp.float32)
    table_pa = table.astype(jnp.float32) + delta
    idx = jnp.pad(region_ids.astype(jnp.int32), (0, _IDX_PAD - _NUM_REGIONS))

    mesh = plsc.VectorSubcoreMesh(core_axis_name="c", subcore_axis_name="s")
    run = functools.partial(
        pl.kernel,
        mesh=mesh,
        out_type=jax.ShapeDtypeStruct((_BATCH, _NUM_REGIONS, _EMB_DIM), jnp.float32),
        scratch_types=[
            pltpu.VMEM((_IDX_PAD,), jnp.int32),
            pltpu.VMEM((_IDX_PAD, _EMB_DIM), jnp.float32),
            pltpu.SemaphoreType.DMA,
        ],
    )(_sc_body)
    return run(table_pa, idx)
